# speedup vs baseline: 1.3168x; 1.3168x over previous
import jax
import jax.numpy as jnp
from jax.experimental import pallas as pl


def kernel(x, table):
    B, S, D = x.shape
    BS = 256

    def body(x_ref, t_ref, o_ref):
        o_ref[...] = x_ref[...] + t_ref[...]

    out = pl.pallas_call(
        body,
        grid=(B, S // BS),
        in_specs=[
            pl.BlockSpec((1, BS, D), lambda b, s: (b, s, 0)),
            pl.BlockSpec((BS, D), lambda b, s: (s, 0)),
        ],
        out_specs=pl.BlockSpec((1, BS, D), lambda b, s: (b, s, 0)),
        out_shape=jax.ShapeDtypeStruct((B, S, D), x.dtype),
    )(x, table)
    return out


# speedup vs baseline: 1.6634x; 1.2632x over previous
import jax
import jax.numpy as jnp
from jax.experimental import pallas as pl


def kernel(x, table):
    B, S, D = x.shape
    BS = 256

    def body(x_ref, t_ref, o_ref):
        o_ref[...] = x_ref[...] + t_ref[...]

    out = pl.pallas_call(
        body,
        grid=(S // BS, B),
        in_specs=[
            pl.BlockSpec((1, BS, D), lambda s, b: (b, s, 0)),
            pl.BlockSpec((BS, D), lambda s, b: (s, 0)),
        ],
        out_specs=pl.BlockSpec((1, BS, D), lambda s, b: (b, s, 0)),
        out_shape=jax.ShapeDtypeStruct((B, S, D), x.dtype),
    )(x, table)
    return out
